# Initial kernel scaffold; baseline (speedup 1.0000x reference)
import jax
import jax.numpy as jnp
from jax.experimental import pallas as pl


def kernel(x, bin_edges, node_keys, graph_emb):
    raise NotImplementedError("write your pallas kernel here")


# speedup vs baseline: 4.2441x; 4.2441x over previous
import functools

import jax
import jax.numpy as jnp
from jax import lax
from jax.experimental import pallas as pl
from jax.experimental.pallas import tpu as pltpu
from jax.experimental.pallas import tpu_sc as plsc

B = 16
S = 512
D = 128
N = 4096
Q = B * S
QT = 128
NW = 32
BPW = Q // NW


def _digitize_body(x_ref, e_ref, out_ref):
    xv = x_ref[...]
    ev = e_ref[...]
    ge = (xv >= ev).astype(jnp.float32)
    out_ref[...] = jnp.sum(ge, axis=0, keepdims=True)


def _argmin_body(d0_ref, d1_ref, d2_ref, nk0_ref, nk1_ref, nk2_ref, out_ref):
    e0 = d0_ref[...] * -2.0
    e1 = d1_ref[...] * -2.0
    e2 = d2_ref[...] * -2.0
    nk0 = nk0_ref[...]
    nk1 = nk1_ref[...]
    nk2 = nk2_ref[...]
    n2 = nk0 * nk0 + nk1 * nk1 + nk2 * nk2
    acc = n2 + nk0 * e0 + nk1 * e1 + nk2 * e2
    vi = acc.astype(jnp.int32)
    iota = lax.broadcasted_iota(jnp.int32, (N, 1), 0)
    c = vi * 4096 + iota
    m = jnp.min(c, axis=0)
    out_ref[...] = (m & 4095).reshape(1, QT)


def _sc_gather_body(idx_hbm, tab_hbm, out_hbm, idx_v, rows_v, sem):
    c = lax.axis_index("c")
    s = lax.axis_index("s")
    wid = s * 2 + c
    base = wid * BPW
    pltpu.sync_copy(idx_hbm.at[pl.ds(base, BPW)], idx_v)
    pltpu.async_copy(tab_hbm.at[idx_v], rows_v, sem).wait()

    @pl.when(wid % 2 == 1)
    def _zero_pad_rows():
        z = jnp.zeros((16,), jnp.float32)
        for r in (BPW - 2, BPW - 1):
            for cc in range(D // 16):
                rows_v[r, pl.ds(cc * 16, 16)] = z

    pltpu.sync_copy(rows_v, out_hbm.at[pl.ds(base, BPW)])


def _make_sc_gather():
    return functools.partial(
        pl.kernel,
        out_type=jax.ShapeDtypeStruct((Q, D), jnp.float32),
        mesh=plsc.VectorSubcoreMesh(core_axis_name="c", subcore_axis_name="s"),
        scratch_types=[
            pltpu.VMEM((BPW,), jnp.int32),
            pltpu.VMEM((BPW, D), jnp.float32),
            pltpu.SemaphoreType.DMA,
        ],
    )(_sc_gather_body)


def kernel(x, bin_edges, node_keys, graph_emb):
    xf = x.reshape(1, Q)
    ev = jnp.concatenate(
        [bin_edges, jnp.full((1,), jnp.inf, jnp.float32)]).reshape(64, 1)

    disc = pl.pallas_call(
        _digitize_body,
        out_shape=jax.ShapeDtypeStruct((1, Q), jnp.float32),
    )(xf, ev)

    df = disc.reshape(Q)
    d1 = jnp.roll(df, -1).reshape(1, Q)
    d2 = jnp.roll(df, -2).reshape(1, Q)

    nid = pl.pallas_call(
        _argmin_body,
        grid=(Q // QT,),
        in_specs=[
            pl.BlockSpec((1, QT), lambda t: (0, t)),
            pl.BlockSpec((1, QT), lambda t: (0, t)),
            pl.BlockSpec((1, QT), lambda t: (0, t)),
            pl.BlockSpec((N, 1), lambda t: (0, 0)),
            pl.BlockSpec((N, 1), lambda t: (0, 0)),
            pl.BlockSpec((N, 1), lambda t: (0, 0)),
        ],
        out_specs=pl.BlockSpec((1, QT), lambda t: (0, t)),
        out_shape=jax.ShapeDtypeStruct((1, Q), jnp.int32),
    )(disc, d1, d2,
      node_keys[:, 0:1], node_keys[:, 1:2], node_keys[:, 2:3])

    out = _make_sc_gather()(nid.reshape(Q), graph_emb)
    return out.reshape(B, S, D)


# speedup vs baseline: 8.1841x; 1.9284x over previous
import functools

import jax
import jax.numpy as jnp
from jax import lax
from jax.experimental import pallas as pl
from jax.experimental.pallas import tpu as pltpu
from jax.experimental.pallas import tpu_sc as plsc

B = 16
S = 512
D = 128
N = 4096
Q = B * S
QT = 128
NSTEPS = Q // QT
NCH = N // 128
RB = 4
NW = 32
BPW = Q // NW


def _prep_body(x_ref, e_ref, nk0_ref, nk1_ref, nk2_ref,
               disc_ref, m0_ref, m1_ref, m2_ref, bi_ref):
    xv = x_ref[...]
    ev = e_ref[...]
    ge = (xv >= ev).astype(jnp.float32)
    disc_ref[...] = jnp.sum(ge, axis=0, keepdims=True)

    nk0 = nk0_ref[...]
    nk1 = nk1_ref[...]
    nk2 = nk2_ref[...]
    n2 = nk0 * nk0 + nk1 * nk1 + nk2 * nk2
    nid = (lax.broadcasted_iota(jnp.int32, (NCH, 128), 0) * 128
           + lax.broadcasted_iota(jnp.int32, (NCH, 128), 1))
    bi = (n2 * 4096.0).astype(jnp.int32) + nid
    m0 = nk0 * -8192.0
    m1 = nk1 * -8192.0
    m2 = nk2 * -8192.0
    for ch in range(NCH):
        sl = pl.ds(ch * 8, 8)
        m0_ref[sl, :] = jnp.broadcast_to(m0[ch:ch + 1, :], (8, 128))
        m1_ref[sl, :] = jnp.broadcast_to(m1[ch:ch + 1, :], (8, 128))
        m2_ref[sl, :] = jnp.broadcast_to(m2[ch:ch + 1, :], (8, 128))
        bi_ref[sl, :] = jnp.broadcast_to(bi[ch:ch + 1, :], (8, 128))


def _argmin_body(dc0_ref, dc1_ref, dc2_ref, m0_ref, m1_ref, m2_ref, bi_ref,
                 out_ref):
    for r0 in range(QT // (8 * RB)):
        d0 = []
        d1 = []
        d2 = []
        cmin = [None] * RB
        for rr in range(RB):
            sl = pl.ds((r0 * RB + rr) * 8, 8)
            d0.append(jnp.broadcast_to(dc0_ref[0, sl, :], (8, 128)))
            d1.append(jnp.broadcast_to(dc1_ref[0, sl, :], (8, 128)))
            d2.append(jnp.broadcast_to(dc2_ref[0, sl, :], (8, 128)))
        for ch in range(NCH):
            sl = pl.ds(ch * 8, 8)
            m0 = m0_ref[sl, :]
            m1 = m1_ref[sl, :]
            m2 = m2_ref[sl, :]
            bi = bi_ref[sl, :]
            for rr in range(RB):
                acc = m0 * d0[rr] + m1 * d1[rr] + m2 * d2[rr]
                c = acc.astype(jnp.int32) + bi
                cmin[rr] = c if cmin[rr] is None else jnp.minimum(cmin[rr], c)
        for rr in range(RB):
            sl = pl.ds((r0 * RB + rr) * 8, 8)
            m = jnp.min(cmin[rr], axis=1, keepdims=True)
            out_ref[0, sl, :] = m & 4095


def _sc_gather_body(idx_hbm, tab_hbm, out_hbm, idx_v, rows_v, sem):
    c = lax.axis_index("c")
    s = lax.axis_index("s")
    wid = s * 2 + c
    base = wid * BPW
    pltpu.sync_copy(idx_hbm.at[pl.ds(base, BPW)], idx_v)
    pltpu.async_copy(tab_hbm.at[idx_v], rows_v, sem).wait()

    @pl.when(wid % 2 == 1)
    def _zero_pad_rows():
        z = jnp.zeros((16,), jnp.float32)
        for r in (BPW - 2, BPW - 1):
            for cc in range(D // 16):
                rows_v[r, pl.ds(cc * 16, 16)] = z

    pltpu.sync_copy(rows_v, out_hbm.at[pl.ds(base, BPW)])


def _make_sc_gather():
    return functools.partial(
        pl.kernel,
        out_type=jax.ShapeDtypeStruct((Q, D), jnp.float32),
        mesh=plsc.VectorSubcoreMesh(core_axis_name="c", subcore_axis_name="s"),
        scratch_types=[
            pltpu.VMEM((BPW,), jnp.int32),
            pltpu.VMEM((BPW, D), jnp.float32),
            pltpu.SemaphoreType.DMA,
        ],
    )(_sc_gather_body)


def kernel(x, bin_edges, node_keys, graph_emb):
    xf = x.reshape(1, Q)
    ev = jnp.concatenate(
        [bin_edges, jnp.full((1,), jnp.inf, jnp.float32)]).reshape(64, 1)

    rep = jax.ShapeDtypeStruct((8 * NCH, 128), jnp.float32)
    disc, m0, m1, m2, bi = pl.pallas_call(
        _prep_body,
        out_shape=(
            jax.ShapeDtypeStruct((1, Q), jnp.float32),
            rep, rep, rep,
            jax.ShapeDtypeStruct((8 * NCH, 128), jnp.int32),
        ),
    )(xf, ev,
      node_keys[:, 0].reshape(NCH, 128),
      node_keys[:, 1].reshape(NCH, 128),
      node_keys[:, 2].reshape(NCH, 128))

    df = disc.reshape(Q)
    dc0 = df.reshape(NSTEPS, QT, 1)
    dc1 = jnp.roll(df, -1).reshape(NSTEPS, QT, 1)
    dc2 = jnp.roll(df, -2).reshape(NSTEPS, QT, 1)

    col = pl.BlockSpec((1, QT, 1), lambda t: (t, 0, 0))
    full = pl.BlockSpec((8 * NCH, 128), lambda t: (0, 0))
    nid = pl.pallas_call(
        _argmin_body,
        grid=(NSTEPS,),
        in_specs=[col, col, col, full, full, full, full],
        out_specs=pl.BlockSpec((1, QT, 1), lambda t: (t, 0, 0)),
        out_shape=jax.ShapeDtypeStruct((NSTEPS, QT, 1), jnp.int32),
    )(dc0, dc1, dc2, m0, m1, m2, bi)

    out = _make_sc_gather()(nid.reshape(Q), graph_emb)
    return out.reshape(B, S, D)


# speedup vs baseline: 9.4996x; 1.1607x over previous
import functools

import jax
import jax.numpy as jnp
from jax import lax
from jax.experimental import pallas as pl
from jax.experimental.pallas import tpu as pltpu
from jax.experimental.pallas import tpu_sc as plsc

B = 16
S = 512
D = 128
N = 4096
Q = B * S
QT = 128
NSTEPS = Q // QT
NCH = N // 128
RB = 4
OFF = 23814.0
NW = 32
BPW = Q // NW


def _prep_body(x_ref, e_ref, nk0_ref, nk1_ref, nk2_ref,
               disc_ref, m0_ref, m1_ref, m2_ref, b_ref):
    xv = x_ref[...]
    ev = e_ref[...]
    ge = (xv >= ev).astype(jnp.float32)
    disc_ref[...] = jnp.sum(ge, axis=0, keepdims=True)

    nk0 = nk0_ref[...]
    nk1 = nk1_ref[...]
    nk2 = nk2_ref[...]
    n2 = nk0 * nk0 + nk1 * nk1 + nk2 * nk2
    chid = lax.broadcasted_iota(jnp.int32, (NCH, 128), 0).astype(jnp.float32)
    b = (n2 + OFF) * 32.0 + chid
    m0 = nk0 * -64.0
    m1 = nk1 * -64.0
    m2 = nk2 * -64.0
    for ch in range(NCH):
        sl = pl.ds(ch * 8, 8)
        m0_ref[sl, :] = jnp.broadcast_to(m0[ch:ch + 1, :], (8, 128))
        m1_ref[sl, :] = jnp.broadcast_to(m1[ch:ch + 1, :], (8, 128))
        m2_ref[sl, :] = jnp.broadcast_to(m2[ch:ch + 1, :], (8, 128))
        b_ref[sl, :] = jnp.broadcast_to(b[ch:ch + 1, :], (8, 128))


def _argmin_body(dc0_ref, dc1_ref, dc2_ref, m0_ref, m1_ref, m2_ref, b_ref,
                 out_ref):
    dcv0 = dc0_ref[0]
    dcv1 = dc1_ref[0]
    dcv2 = dc2_ref[0]
    lane = lax.broadcasted_iota(jnp.int32, (8, 128), 1).astype(jnp.float32)
    for r0 in range(QT // (8 * RB)):
        d0 = []
        d1 = []
        d2 = []
        cmin = [None] * RB
        for rr in range(RB):
            q = r0 * RB + rr
            d0.append(jnp.broadcast_to(dcv0[:, q:q + 1], (8, 128)))
            d1.append(jnp.broadcast_to(dcv1[:, q:q + 1], (8, 128)))
            d2.append(jnp.broadcast_to(dcv2[:, q:q + 1], (8, 128)))
        for ch in range(NCH):
            sl = pl.ds(ch * 8, 8)
            m0 = m0_ref[sl, :]
            m1 = m1_ref[sl, :]
            m2 = m2_ref[sl, :]
            b = b_ref[sl, :]
            for rr in range(RB):
                c = b + m0 * d0[rr] + m1 * d1[rr] + m2 * d2[rr]
                cmin[rr] = c if cmin[rr] is None else jnp.minimum(cmin[rr], c)
        for rr in range(RB):
            q = r0 * RB + rr
            cm = cmin[rr]
            mv = jnp.min(cm, axis=1, keepdims=True)
            lv = jnp.min(jnp.where(cm == mv, lane, 128.0),
                         axis=1, keepdims=True)
            nid = ((mv.astype(jnp.int32) & 31) * 128
                   + lv.astype(jnp.int32))
            out_ref[0, :, q:q + 1] = nid


def _sc_gather_body(idx_hbm, tab_hbm, out_hbm, idx_v, rows_v, sem):
    c = lax.axis_index("c")
    s = lax.axis_index("s")
    wid = s * 2 + c
    base = wid * BPW
    pltpu.sync_copy(idx_hbm.at[pl.ds(base, BPW)], idx_v)
    pltpu.async_copy(tab_hbm.at[idx_v], rows_v, sem).wait()

    @pl.when(wid % 2 == 1)
    def _zero_pad_rows():
        z = jnp.zeros((16,), jnp.float32)
        for r in (BPW - 2, BPW - 1):
            for cc in range(D // 16):
                rows_v[r, pl.ds(cc * 16, 16)] = z

    pltpu.sync_copy(rows_v, out_hbm.at[pl.ds(base, BPW)])


def _make_sc_gather():
    return functools.partial(
        pl.kernel,
        out_type=jax.ShapeDtypeStruct((Q, D), jnp.float32),
        mesh=plsc.VectorSubcoreMesh(core_axis_name="c", subcore_axis_name="s"),
        scratch_types=[
            pltpu.VMEM((BPW,), jnp.int32),
            pltpu.VMEM((BPW, D), jnp.float32),
            pltpu.SemaphoreType.DMA,
        ],
    )(_sc_gather_body)


def kernel(x, bin_edges, node_keys, graph_emb):
    xf = x.reshape(1, Q)
    ev = jnp.concatenate(
        [bin_edges, jnp.full((1,), jnp.inf, jnp.float32)]).reshape(64, 1)

    rep = jax.ShapeDtypeStruct((8 * NCH, 128), jnp.float32)
    disc, m0, m1, m2, bv = pl.pallas_call(
        _prep_body,
        out_shape=(
            jax.ShapeDtypeStruct((1, Q), jnp.float32),
            rep, rep, rep, rep,
        ),
    )(xf, ev,
      node_keys[:, 0].reshape(NCH, 128),
      node_keys[:, 1].reshape(NCH, 128),
      node_keys[:, 2].reshape(NCH, 128))

    df = disc.reshape(Q)
    dc0 = df.reshape(NSTEPS, 16, 8).transpose(0, 2, 1)
    dc1 = jnp.roll(df, -1).reshape(NSTEPS, 16, 8).transpose(0, 2, 1)
    dc2 = jnp.roll(df, -2).reshape(NSTEPS, 16, 8).transpose(0, 2, 1)

    col = pl.BlockSpec((1, 8, 16), lambda t: (t, 0, 0))
    full = pl.BlockSpec((8 * NCH, 128), lambda t: (0, 0))
    nid = pl.pallas_call(
        _argmin_body,
        grid=(NSTEPS,),
        in_specs=[col, col, col, full, full, full, full],
        out_specs=pl.BlockSpec((1, 8, 16), lambda t: (t, 0, 0)),
        out_shape=jax.ShapeDtypeStruct((NSTEPS, 8, 16), jnp.int32),
    )(dc0, dc1, dc2, m0, m1, m2, bv)

    nid_flat = nid.transpose(0, 2, 1).reshape(Q)
    out = _make_sc_gather()(nid_flat, graph_emb)
    return out.reshape(B, S, D)


# speedup vs baseline: 13.8791x; 1.4610x over previous
import functools

import jax
import jax.numpy as jnp
from jax import lax
from jax.experimental import pallas as pl
from jax.experimental.pallas import tpu as pltpu
from jax.experimental.pallas import tpu_sc as plsc

B = 16
S = 512
D = 128
N = 4096
NE = 63
Q = B * S
QT = 1024
NSTEPS = Q // QT
LQ = QT // 8
NCH = N // 128
OFF = 23814.0
NW = 32
BPW = Q // NW


def _tc_body(x_ref, e_ref, nk0_ref, nk1_ref, nk2_ref, out_ref,
             dc0_s, dc1_s, dc2_s, m0_s, m1_s, m2_s, b_s):
    t = pl.program_id(0)

    @pl.when(t == 0)
    def _prep():
        xv = x_ref[...]
        acc = jnp.zeros((Q // 128, 128), jnp.float32)
        for j in range(NE):
            acc += (xv >= e_ref[0, j]).astype(jnp.float32)
        dc0_s[...] = acc
        nxt = pltpu.roll(acc, Q // 128 - 1, 0)
        dc1_s[...] = jnp.concatenate([acc[:, 1:], nxt[:, :1]], axis=1)
        dc2_s[...] = jnp.concatenate([acc[:, 2:], nxt[:, :2]], axis=1)

        nk0 = nk0_ref[...]
        nk1 = nk1_ref[...]
        nk2 = nk2_ref[...]
        n2 = nk0 * nk0 + nk1 * nk1 + nk2 * nk2
        chid = lax.broadcasted_iota(
            jnp.int32, (NCH, 128), 0).astype(jnp.float32)
        bv = (n2 + OFF) * 32.0 + chid
        m0 = nk0 * -64.0
        m1 = nk1 * -64.0
        m2 = nk2 * -64.0
        for ch in range(NCH):
            sl = pl.ds(ch * 8, 8)
            m0_s[sl, :] = jnp.broadcast_to(m0[ch:ch + 1, :], (8, 128))
            m1_s[sl, :] = jnp.broadcast_to(m1[ch:ch + 1, :], (8, 128))
            m2_s[sl, :] = jnp.broadcast_to(m2[ch:ch + 1, :], (8, 128))
            b_s[sl, :] = jnp.broadcast_to(bv[ch:ch + 1, :], (8, 128))

    rows = pl.ds(t * 8, 8)
    dcv0 = dc0_s[rows, :]
    dcv1 = dc1_s[rows, :]
    dcv2 = dc2_s[rows, :]
    lane = lax.broadcasted_iota(jnp.int32, (8, 128), 1).astype(jnp.float32)
    d0 = []
    d1 = []
    d2 = []
    cmin = [None] * LQ
    for q in range(LQ):
        d0.append(jnp.broadcast_to(dcv0[:, q:q + 1], (8, 128)))
        d1.append(jnp.broadcast_to(dcv1[:, q:q + 1], (8, 128)))
        d2.append(jnp.broadcast_to(dcv2[:, q:q + 1], (8, 128)))
    for ch in range(NCH):
        sl = pl.ds(ch * 8, 8)
        m0 = m0_s[sl, :]
        m1 = m1_s[sl, :]
        m2 = m2_s[sl, :]
        bv = b_s[sl, :]
        for q in range(LQ):
            c = bv + m0 * d0[q] + m1 * d1[q] + m2 * d2[q]
            cmin[q] = c if cmin[q] is None else jnp.minimum(cmin[q], c)
    for q in range(LQ):
        cm = cmin[q]
        mv = jnp.min(cm, axis=1, keepdims=True)
        lv = jnp.min(jnp.where(cm == mv, lane, 128.0),
                     axis=1, keepdims=True)
        nid = (mv.astype(jnp.int32) & 31) * 128 + lv.astype(jnp.int32)
        out_ref[0, :, q:q + 1] = nid


def _sc_gather_body(idx_hbm, tab_hbm, out_hbm, idx_v, rows_v, sem):
    c = lax.axis_index("c")
    s = lax.axis_index("s")
    wid = s * 2 + c
    base = wid * BPW
    pltpu.sync_copy(idx_hbm.at[pl.ds(base, BPW)], idx_v)
    pltpu.async_copy(tab_hbm.at[idx_v], rows_v, sem).wait()

    @pl.when(wid % 2 == 1)
    def _zero_pad_rows():
        z = jnp.zeros((16,), jnp.float32)
        for r in (BPW - 2, BPW - 1):
            for cc in range(D // 16):
                rows_v[r, pl.ds(cc * 16, 16)] = z

    pltpu.sync_copy(rows_v, out_hbm.at[pl.ds(base, BPW)])


def _make_sc_gather():
    return functools.partial(
        pl.kernel,
        out_type=jax.ShapeDtypeStruct((Q, D), jnp.float32),
        mesh=plsc.VectorSubcoreMesh(core_axis_name="c", subcore_axis_name="s"),
        scratch_types=[
            pltpu.VMEM((BPW,), jnp.int32),
            pltpu.VMEM((BPW, D), jnp.float32),
            pltpu.SemaphoreType.DMA,
        ],
    )(_sc_gather_body)


def kernel(x, bin_edges, node_keys, graph_emb):
    x2d = x.reshape(Q // 128, 128)
    ev = bin_edges.reshape(1, NE)

    sc64 = pltpu.VMEM((Q // 128, 128), jnp.float32)
    screp = pltpu.VMEM((8 * NCH, 128), jnp.float32)
    nid = pl.pallas_call(
        _tc_body,
        grid=(NSTEPS,),
        in_specs=[
            pl.BlockSpec((Q // 128, 128), lambda t: (0, 0)),
            pl.BlockSpec(memory_space=pltpu.SMEM),
            pl.BlockSpec((NCH, 128), lambda t: (0, 0)),
            pl.BlockSpec((NCH, 128), lambda t: (0, 0)),
            pl.BlockSpec((NCH, 128), lambda t: (0, 0)),
        ],
        out_specs=pl.BlockSpec((1, 8, LQ), lambda t: (t, 0, 0)),
        out_shape=jax.ShapeDtypeStruct((NSTEPS, 8, LQ), jnp.int32),
        scratch_shapes=[sc64, sc64, sc64, screp, screp, screp, screp],
    )(x2d, ev,
      node_keys[:, 0].reshape(NCH, 128),
      node_keys[:, 1].reshape(NCH, 128),
      node_keys[:, 2].reshape(NCH, 128))

    out = _make_sc_gather()(nid.reshape(Q), graph_emb)
    return out.reshape(B, S, D)


# speedup vs baseline: 13.9531x; 1.0053x over previous
import functools

import jax
import jax.numpy as jnp
from jax import lax
from jax.experimental import pallas as pl
from jax.experimental.pallas import tpu as pltpu
from jax.experimental.pallas import tpu_sc as plsc

B = 16
S = 512
D = 128
N = 4096
NE = 63
Q = B * S
QT = 1024
NSTEPS = Q // QT
LQ = QT // 8
NCH = N // 128
OFF = 23814.0
NW = 32
BPW = Q // NW


def _tc_body(x_ref, e_ref, nk0_ref, nk1_ref, nk2_ref, out_ref,
             dc0_s, dc1_s, dc2_s, m0_s, m1_s, m2_s, b_s):
    t = pl.program_id(0)

    @pl.when(t == 0)
    def _prep():
        xv = x_ref[...]
        acc = jnp.zeros((Q // 128, 128), jnp.float32)
        for j in range(NE):
            acc += (xv >= e_ref[0, j]).astype(jnp.float32)
        dc0_s[...] = acc
        nxt = pltpu.roll(acc, Q // 128 - 1, 0)
        dc1_s[...] = jnp.concatenate([acc[:, 1:], nxt[:, :1]], axis=1)
        dc2_s[...] = jnp.concatenate([acc[:, 2:], nxt[:, :2]], axis=1)

        nk0 = nk0_ref[...]
        nk1 = nk1_ref[...]
        nk2 = nk2_ref[...]
        n2 = nk0 * nk0 + nk1 * nk1 + nk2 * nk2
        chid = lax.broadcasted_iota(
            jnp.int32, (NCH, 128), 0).astype(jnp.float32)
        bv = (n2 + OFF) * 32.0 + chid
        m0 = nk0 * -64.0
        m1 = nk1 * -64.0
        m2 = nk2 * -64.0
        for ch in range(NCH):
            sl = pl.ds(ch * 8, 8)
            m0_s[sl, :] = jnp.broadcast_to(m0[ch:ch + 1, :], (8, 128))
            m1_s[sl, :] = jnp.broadcast_to(m1[ch:ch + 1, :], (8, 128))
            m2_s[sl, :] = jnp.broadcast_to(m2[ch:ch + 1, :], (8, 128))
            b_s[sl, :] = jnp.broadcast_to(bv[ch:ch + 1, :], (8, 128))

    rows = pl.ds(t * 8, 8)
    dcv0 = dc0_s[rows, :]
    dcv1 = dc1_s[rows, :]
    dcv2 = dc2_s[rows, :]
    lane = lax.broadcasted_iota(jnp.int32, (8, 128), 1).astype(jnp.float32)
    d0 = []
    d1 = []
    d2 = []
    cmin = [None] * LQ
    for q in range(LQ):
        d0.append(jnp.broadcast_to(dcv0[:, q:q + 1], (8, 128)))
        d1.append(jnp.broadcast_to(dcv1[:, q:q + 1], (8, 128)))
        d2.append(jnp.broadcast_to(dcv2[:, q:q + 1], (8, 128)))
    for ch in range(NCH):
        sl = pl.ds(ch * 8, 8)
        m0 = m0_s[sl, :]
        m1 = m1_s[sl, :]
        m2 = m2_s[sl, :]
        bv = b_s[sl, :]
        for q in range(LQ):
            c = bv + m0 * d0[q] + m1 * d1[q] + m2 * d2[q]
            cmin[q] = c if cmin[q] is None else jnp.minimum(cmin[q], c)
    for q in range(LQ):
        cm = cmin[q]
        mv = jnp.min(cm, axis=1, keepdims=True)
        lv = jnp.min(jnp.where(cm == mv, lane, 128.0),
                     axis=1, keepdims=True)
        nid = (mv.astype(jnp.int32) & 31) * 128 + lv.astype(jnp.int32)
        out_ref[0, :, q:q + 1] = nid


HPW = BPW // 2


def _sc_gather_body(idx_hbm, tab_hbm, out_hbm,
                    idx_v0, idx_v1, rows_v0, rows_v1,
                    gsem0, gsem1, ssem0, ssem1):
    c = lax.axis_index("c")
    s = lax.axis_index("s")
    wid = s * 2 + c
    base = wid * BPW
    pltpu.sync_copy(idx_hbm.at[pl.ds(base, HPW)], idx_v0)
    g0 = pltpu.async_copy(tab_hbm.at[idx_v0], rows_v0, gsem0)
    pltpu.sync_copy(idx_hbm.at[pl.ds(base + HPW, HPW)], idx_v1)
    g1 = pltpu.async_copy(tab_hbm.at[idx_v1], rows_v1, gsem1)
    g0.wait()
    s0 = pltpu.async_copy(rows_v0, out_hbm.at[pl.ds(base, HPW)], ssem0)
    g1.wait()

    @pl.when(wid % 2 == 1)
    def _zero_pad_rows():
        z = jnp.zeros((16,), jnp.float32)
        for r in (HPW - 2, HPW - 1):
            for cc in range(D // 16):
                rows_v1[r, pl.ds(cc * 16, 16)] = z

    s1 = pltpu.async_copy(rows_v1, out_hbm.at[pl.ds(base + HPW, HPW)], ssem1)
    s0.wait()
    s1.wait()


def _make_sc_gather():
    return functools.partial(
        pl.kernel,
        out_type=jax.ShapeDtypeStruct((Q, D), jnp.float32),
        mesh=plsc.VectorSubcoreMesh(core_axis_name="c", subcore_axis_name="s"),
        scratch_types=[
            pltpu.VMEM((HPW,), jnp.int32),
            pltpu.VMEM((HPW,), jnp.int32),
            pltpu.VMEM((HPW, D), jnp.float32),
            pltpu.VMEM((HPW, D), jnp.float32),
            pltpu.SemaphoreType.DMA,
            pltpu.SemaphoreType.DMA,
            pltpu.SemaphoreType.DMA,
            pltpu.SemaphoreType.DMA,
        ],
    )(_sc_gather_body)


def kernel(x, bin_edges, node_keys, graph_emb):
    x2d = x.reshape(Q // 128, 128)
    ev = bin_edges.reshape(1, NE)

    sc64 = pltpu.VMEM((Q // 128, 128), jnp.float32)
    screp = pltpu.VMEM((8 * NCH, 128), jnp.float32)
    nid = pl.pallas_call(
        _tc_body,
        grid=(NSTEPS,),
        in_specs=[
            pl.BlockSpec((Q // 128, 128), lambda t: (0, 0)),
            pl.BlockSpec(memory_space=pltpu.SMEM),
            pl.BlockSpec((NCH, 128), lambda t: (0, 0)),
            pl.BlockSpec((NCH, 128), lambda t: (0, 0)),
            pl.BlockSpec((NCH, 128), lambda t: (0, 0)),
        ],
        out_specs=pl.BlockSpec((1, 8, LQ), lambda t: (t, 0, 0)),
        out_shape=jax.ShapeDtypeStruct((NSTEPS, 8, LQ), jnp.int32),
        scratch_shapes=[sc64, sc64, sc64, screp, screp, screp, screp],
    )(x2d, ev,
      node_keys[:, 0].reshape(NCH, 128),
      node_keys[:, 1].reshape(NCH, 128),
      node_keys[:, 2].reshape(NCH, 128))

    out = _make_sc_gather()(nid.reshape(Q), graph_emb)
    return out.reshape(B, S, D)
